# initial kernel scaffold (unmeasured)
import jax
import jax.numpy as jnp
from jax import lax
from jax.experimental import pallas as pl
from jax.experimental.pallas import tpu as pltpu

N_DEV = 4


def kernel(x, w_mat, scale_x, scale_w):
    m_glob, k_loc = x.shape
    k_glob, n = w_mat.shape
    m_loc = m_glob // N_DEV
    print(f"[kernel trace] x dtype={x.dtype}, w dtype={w_mat.dtype}, "
          f"sx={scale_x.dtype} {scale_x.shape}, sw={scale_w.dtype}")

    def body(x_ref, w_ref, sx_ref, sw_ref, out_ref, xg_ref, send_sems, recv_sems):
        my = lax.axis_index("i")

        barrier_sem = pltpu.get_barrier_semaphore()
        for d in range(1, N_DEV):
            peer = lax.rem(my + d, N_DEV)
            pl.semaphore_signal(
                barrier_sem, inc=1,
                device_id=(peer,), device_id_type=pl.DeviceIdType.MESH,
            )
        pl.semaphore_wait(barrier_sem, N_DEV - 1)

        rdmas = []
        for d in range(1, N_DEV):
            peer = lax.rem(my + d, N_DEV)
            rdma = pltpu.make_async_remote_copy(
                src_ref=x_ref.at[pl.ds(peer * m_loc, m_loc), :],
                dst_ref=xg_ref.at[d - 1],
                send_sem=send_sems.at[d - 1],
                recv_sem=recv_sems.at[d - 1],
                device_id=(peer,),
                device_id_type=pl.DeviceIdType.MESH,
            )
            rdma.start()
            rdmas.append(rdma)

        scale = sx_ref[0] * sw_ref[0]

        x_own = x_ref[pl.ds(my * m_loc, m_loc), :]
        w_own = w_ref[pl.ds(my * k_loc, k_loc), :]
        out_ref[:, :] = jnp.dot(x_own, w_own, preferred_element_type=jnp.float32)

        for d in range(1, N_DEV):
            rdmas[d - 1].wait()
            origin = lax.rem(my - d + N_DEV, N_DEV)
            w_blk = w_ref[pl.ds(origin * k_loc, k_loc), :]
            out_ref[:, :] += jnp.dot(
                xg_ref[d - 1], w_blk, preferred_element_type=jnp.float32
            )

        out_ref[:, :] = jnp.maximum(out_ref[:, :] * scale, 0.0)

    return pl.pallas_call(
        body,
        out_shape=jax.ShapeDtypeStruct((m_loc, n), jnp.float32),
        in_specs=[
            pl.BlockSpec(memory_space=pltpu.VMEM),
            pl.BlockSpec(memory_space=pltpu.VMEM),
            pl.BlockSpec(memory_space=pltpu.SMEM),
            pl.BlockSpec(memory_space=pltpu.SMEM),
        ],
        out_specs=pl.BlockSpec(memory_space=pltpu.VMEM),
        scratch_shapes=[
            pltpu.VMEM((N_DEV - 1, m_loc, k_loc), x.dtype),
            pltpu.SemaphoreType.DMA((N_DEV - 1,)),
            pltpu.SemaphoreType.DMA((N_DEV - 1,)),
        ],
        compiler_params=pltpu.CompilerParams(collective_id=0),
    )(x, w_mat, scale_x, scale_w)


# baseline (device time: 141321 ns/iter reference)
import jax
import jax.numpy as jnp
from jax import lax
from jax.experimental import pallas as pl
from jax.experimental.pallas import tpu as pltpu

N_DEV = 4
N_TILE = 512


def kernel(x, w_mat, scale_x, scale_w):
    m_glob, k_loc = x.shape
    k_glob, n = w_mat.shape
    m_loc = m_glob // N_DEV
    n_steps = n // N_TILE

    def body(x_hbm, w_ref, sx_ref, sw_ref, out_ref,
             staging, xsend, xg, recv_buf, copy_sem, send_sems, recv_sems):
        j = pl.program_id(0)
        my = lax.axis_index("i")

        @pl.when(j == 0)
        def _():
            for c in range(N_DEV):
                cp = pltpu.make_async_copy(
                    x_hbm.at[pl.ds(c * m_loc, m_loc), :], staging, copy_sem
                )
                cp.start()
                cp.wait()
                xsend[pl.ds(c * m_loc, m_loc), :] = (
                    staging[:, :].astype(jnp.float8_e4m3fn)
                )

                @pl.when(my == c)
                def _():
                    xg[:, pl.ds(my * k_loc, k_loc)] = (
                        staging[:, :].astype(jnp.bfloat16)
                    )

            barrier_sem = pltpu.get_barrier_semaphore()
            for d in range(1, N_DEV):
                peer = lax.rem(my + d, N_DEV)
                pl.semaphore_signal(
                    barrier_sem, inc=1,
                    device_id=(peer,), device_id_type=pl.DeviceIdType.MESH,
                )
            pl.semaphore_wait(barrier_sem, N_DEV - 1)

            rdmas = []
            for d in range(1, N_DEV):
                peer = lax.rem(my + d, N_DEV)
                rdma = pltpu.make_async_remote_copy(
                    src_ref=xsend.at[pl.ds(peer * m_loc, m_loc), :],
                    dst_ref=recv_buf.at[d - 1],
                    send_sem=send_sems.at[d - 1],
                    recv_sem=recv_sems.at[d - 1],
                    device_id=(peer,),
                    device_id_type=pl.DeviceIdType.MESH,
                )
                rdma.start()
                rdmas.append(rdma)

            for d in range(1, N_DEV):
                rdmas[d - 1].wait()
                origin = lax.rem(my - d + N_DEV, N_DEV)
                xg[:, pl.ds(origin * k_loc, k_loc)] = (
                    recv_buf[d - 1].astype(jnp.bfloat16)
                )

        scale = sx_ref[0] * sw_ref[0]
        wb = w_ref[:, :].astype(jnp.bfloat16)
        acc = jnp.dot(xg[:, :], wb, preferred_element_type=jnp.float32)
        out_ref[:, :] = jnp.maximum(acc * scale, 0.0)

    return pl.pallas_call(
        body,
        grid=(n_steps,),
        out_shape=jax.ShapeDtypeStruct((m_loc, n), jnp.float32),
        in_specs=[
            pl.BlockSpec(memory_space=pltpu.MemorySpace.HBM),
            pl.BlockSpec((k_glob, N_TILE), lambda j: (0, j)),
            pl.BlockSpec(memory_space=pltpu.SMEM),
            pl.BlockSpec(memory_space=pltpu.SMEM),
        ],
        out_specs=pl.BlockSpec((m_loc, N_TILE), lambda j: (0, j)),
        scratch_shapes=[
            pltpu.VMEM((m_loc, k_loc), jnp.float32),
            pltpu.VMEM((m_glob, k_loc), jnp.float8_e4m3fn),
            pltpu.VMEM((m_loc, k_glob), jnp.bfloat16),
            pltpu.VMEM((N_DEV - 1, m_loc, k_loc), jnp.float8_e4m3fn),
            pltpu.SemaphoreType.DMA,
            pltpu.SemaphoreType.DMA((N_DEV - 1,)),
            pltpu.SemaphoreType.DMA((N_DEV - 1,)),
        ],
        compiler_params=pltpu.CompilerParams(
            collective_id=0,
            dimension_semantics=("arbitrary",),
            vmem_limit_bytes=48 * 1024 * 1024,
        ),
    )(x, w_mat, scale_x, scale_w)


# device time: 119499 ns/iter; 1.1826x vs baseline; 1.1826x over previous
import jax
import jax.numpy as jnp
from jax import lax
from jax.experimental import pallas as pl
from jax.experimental.pallas import tpu as pltpu

N_DEV = 4
N_TILE = 512


def kernel(x, w_mat, scale_x, scale_w):
    m_glob, k_loc = x.shape
    k_glob, n = w_mat.shape
    m_loc = m_glob // N_DEV
    n_steps = n // N_TILE

    def body(x_hbm, w_ref, sx_ref, sw_ref, out_ref,
             staging, xsend, xg, recv_buf, copy_sem, send_sems, recv_sems):
        j = pl.program_id(0)
        my = lax.axis_index("i")

        @pl.when(j == 0)
        def _():
            for c in range(N_DEV):
                cp = pltpu.make_async_copy(
                    x_hbm.at[pl.ds(c * m_loc, m_loc), :], staging, copy_sem
                )
                cp.start()
                cp.wait()
                xsend[pl.ds(c * m_loc, m_loc), :] = (
                    staging[:, :].astype(jnp.float8_e4m3fn)
                )

                @pl.when(my == c)
                def _():
                    xg[:, pl.ds(my * k_loc, k_loc)] = (
                        staging[:, :].astype(jnp.float8_e4m3fn)
                    )

            barrier_sem = pltpu.get_barrier_semaphore()
            for d in range(1, N_DEV):
                peer = lax.rem(my + d, N_DEV)
                pl.semaphore_signal(
                    barrier_sem, inc=1,
                    device_id=(peer,), device_id_type=pl.DeviceIdType.MESH,
                )
            pl.semaphore_wait(barrier_sem, N_DEV - 1)

            rdmas = []
            for d in range(1, N_DEV):
                peer = lax.rem(my + d, N_DEV)
                rdma = pltpu.make_async_remote_copy(
                    src_ref=xsend.at[pl.ds(peer * m_loc, m_loc), :],
                    dst_ref=recv_buf.at[d - 1],
                    send_sem=send_sems.at[d - 1],
                    recv_sem=recv_sems.at[d - 1],
                    device_id=(peer,),
                    device_id_type=pl.DeviceIdType.MESH,
                )
                rdma.start()
                rdmas.append(rdma)

            for d in range(1, N_DEV):
                rdmas[d - 1].wait()
                origin = lax.rem(my - d + N_DEV, N_DEV)
                xg[:, pl.ds(origin * k_loc, k_loc)] = recv_buf[d - 1]

        scale = sx_ref[0] * sw_ref[0]
        wb = w_ref[:, :].astype(jnp.float8_e4m3fn)
        acc = jnp.dot(xg[:, :], wb, preferred_element_type=jnp.float32)
        out_ref[:, :] = jnp.maximum(acc * scale, 0.0)

    return pl.pallas_call(
        body,
        grid=(n_steps,),
        out_shape=jax.ShapeDtypeStruct((m_loc, n), jnp.float32),
        in_specs=[
            pl.BlockSpec(memory_space=pltpu.MemorySpace.HBM),
            pl.BlockSpec((k_glob, N_TILE), lambda j: (0, j)),
            pl.BlockSpec(memory_space=pltpu.SMEM),
            pl.BlockSpec(memory_space=pltpu.SMEM),
        ],
        out_specs=pl.BlockSpec((m_loc, N_TILE), lambda j: (0, j)),
        scratch_shapes=[
            pltpu.VMEM((m_loc, k_loc), jnp.float32),
            pltpu.VMEM((m_glob, k_loc), jnp.float8_e4m3fn),
            pltpu.VMEM((m_loc, k_glob), jnp.float8_e4m3fn),
            pltpu.VMEM((N_DEV - 1, m_loc, k_loc), jnp.float8_e4m3fn),
            pltpu.SemaphoreType.DMA,
            pltpu.SemaphoreType.DMA((N_DEV - 1,)),
            pltpu.SemaphoreType.DMA((N_DEV - 1,)),
        ],
        compiler_params=pltpu.CompilerParams(
            collective_id=0,
            dimension_semantics=("arbitrary",),
            vmem_limit_bytes=48 * 1024 * 1024,
        ),
    )(x, w_mat, scale_x, scale_w)
